# baseline (speedup 1.0000x reference)
import functools

import jax
import jax.numpy as jnp
from jax import lax
from jax.experimental import pallas as pl
from jax.experimental.pallas import tpu as pltpu
from jax.experimental.pallas import tpu_sc as plsc

B, N, K, H = 16, 1024, 4, 128
R = 512
T = B * N
NC, NS = 2, 16
NW = NC * NS
TPW = T // NW

G_TOK = 128
NG = TPW // G_TOK


def _sc_body(idx_hbm, table_hbm, ent_hbm, idx_v,
             e0, e1, b0, b1, b2, b3, b4,
             se0, se1, sb0, sb1, sb2, sb3, sb4, ss0, ss1):
    wid = lax.axis_index("s") * NC + lax.axis_index("c")
    pltpu.sync_copy(idx_hbm.at[pl.ds(wid * (TPW * K // 128), TPW * K // 128)],
                    idx_v)

    def gath(g, k, buf, sem):
        pltpu.make_async_copy(table_hbm.at[idx_v.at[4 * g + k]], buf,
                              sem).start()

    def wait(buf, sem):
        pltpu.make_async_copy(table_hbm.at[idx_v.at[0]], buf, sem).wait()

    def reduce(e, a, b, c):
        def red(u, _):
            for dt in range(4):
                t = 4 * u + dt
                for j in range(H // 16):
                    s = pl.ds(j * 16, 16)
                    e[t, s] = e[t, s] + a[t, s] + b[t, s] + c[t, s]
            return 0
        lax.fori_loop(0, G_TOK // 4, red, 0)

    def store(g, e, sem):
        pltpu.make_async_copy(
            e, ent_hbm.at[pl.ds(wid * TPW + g * G_TOK, G_TOK)], sem).start()

    def swait(e, g, sem):
        pltpu.make_async_copy(
            e, ent_hbm.at[pl.ds(wid * TPW + g * G_TOK, G_TOK)], sem).wait()

    gath(0, 0, e0, se0)
    gath(0, 1, b0, sb0)
    gath(0, 2, b1, sb1)
    gath(0, 3, b2, sb2)
    wait(e0, se0); wait(b0, sb0); wait(b1, sb1); wait(b2, sb2)
    gath(1, 0, e1, se1)
    gath(1, 1, b3, sb3)
    gath(1, 2, b4, sb4)
    reduce(e0, b0, b1, b2)
    store(0, e0, ss0)
    gath(1, 3, b0, sb0)
    gath(2, 1, b1, sb1)
    gath(2, 2, b2, sb2)
    wait(e1, se1); wait(b3, sb3); wait(b4, sb4); wait(b0, sb0)
    reduce(e1, b3, b4, b0)
    store(1, e1, ss1)
    swait(e0, 0, ss0)
    gath(2, 0, e0, se0)
    gath(2, 3, b3, sb3)
    gath(3, 1, b4, sb4)
    wait(e0, se0); wait(b1, sb1); wait(b2, sb2); wait(b3, sb3)
    reduce(e0, b1, b2, b3)
    store(2, e0, ss0)
    gath(3, 2, b0, sb0)
    gath(3, 3, b1, sb1)
    swait(e1, 1, ss1)
    gath(3, 0, e1, se1)
    wait(e1, se1); wait(b4, sb4); wait(b0, sb0); wait(b1, sb1)
    reduce(e1, b4, b0, b1)
    store(3, e1, ss1)
    swait(e0, 2, ss0)
    swait(e1, 3, ss1)


_sc_call = functools.partial(
    pl.kernel,
    mesh=plsc.VectorSubcoreMesh(core_axis_name="c", subcore_axis_name="s"),
    out_type=jax.ShapeDtypeStruct((T, H), jnp.float32),
    scratch_types=[
        pltpu.VMEM((TPW * K // 128, 128), jnp.int32),
        pltpu.VMEM((G_TOK, H), jnp.float32),
        pltpu.VMEM((G_TOK, H), jnp.float32),
        pltpu.VMEM((G_TOK, H), jnp.float32),
        pltpu.VMEM((G_TOK, H), jnp.float32),
        pltpu.VMEM((G_TOK, H), jnp.float32),
        pltpu.VMEM((G_TOK, H), jnp.float32),
        pltpu.VMEM((G_TOK, H), jnp.float32),
        pltpu.SemaphoreType.DMA,
        pltpu.SemaphoreType.DMA,
        pltpu.SemaphoreType.DMA,
        pltpu.SemaphoreType.DMA,
        pltpu.SemaphoreType.DMA,
        pltpu.SemaphoreType.DMA,
        pltpu.SemaphoreType.DMA,
        pltpu.SemaphoreType.DMA,
        pltpu.SemaphoreType.DMA,
    ],
)(_sc_body)


TBLK = 512


def _rel_body(rt_ref, fw_ref, link_ref, out_ref, w2_ref):
    @pl.when(pl.program_id(0) == 0)
    def _():
        w2_ref[...] = jnp.dot(rt_ref[...], fw_ref[...],
                              preferred_element_type=jnp.float32)

    ls = (link_ref[:, 0, :] + link_ref[:, 1, :]
          + link_ref[:, 2, :] + link_ref[:, 3, :])
    out_ref[...] = jnp.dot(ls, w2_ref[...],
                           preferred_element_type=jnp.float32
                           ).astype(jnp.bfloat16)


FB = 8


def _fin_body(rel_ref, ent_ref, gt_ref, g_ref, b_ref, out_ref):
    acc = rel_ref[...].astype(jnp.float32) + ent_ref[...]
    mu = jnp.mean(acc, axis=-1, keepdims=True)
    d = acc - mu
    var = jnp.mean(d * d, axis=-1, keepdims=True)
    y = d * lax.rsqrt(var + 1e-6) * g_ref[...] + b_ref[...]
    out_ref[1:, :, :] = jnp.transpose(y, (1, 0, 2))
    out_ref[0:1, :, :] = jnp.broadcast_to(
        gt_ref[...].reshape(1, 1, H), (1, FB, H))


def kernel(x, in_degree, out_degree, link, length, entity_table,
           in_deg_table, out_deg_table, rel_table, ffn_W,
           ln_gamma, ln_beta, graph_token):
    idx = (x.astype(jnp.int32)
           .reshape(B, N // 128, 128, K)
           .transpose(0, 1, 3, 2)
           .reshape(T * K // 128, 128))
    link_flat = link.reshape(T, K, R)
    ent = _sc_call(idx, entity_table)

    rel = pl.pallas_call(
        _rel_body,
        grid=(T // TBLK,),
        in_specs=[
            pl.BlockSpec((R, H), lambda i: (0, 0)),
            pl.BlockSpec((H, H), lambda i: (0, 0)),
            pl.BlockSpec((TBLK, K, R), lambda i: (i, 0, 0)),
        ],
        out_specs=pl.BlockSpec((TBLK, H), lambda i: (i, 0)),
        out_shape=jax.ShapeDtypeStruct((T, H), jnp.bfloat16),
        scratch_shapes=[pltpu.VMEM((R, H), jnp.float32)],
    )(rel_table, ffn_W, link_flat)

    g2 = ln_gamma.reshape(1, H)
    b2 = ln_beta.reshape(1, H)
    out_t = pl.pallas_call(
        _fin_body,
        grid=(B // FB,),
        in_specs=[
            pl.BlockSpec((FB, N, H), lambda i: (i, 0, 0)),
            pl.BlockSpec((FB, N, H), lambda i: (i, 0, 0)),
            pl.BlockSpec((1, H), lambda i: (0, 0)),
            pl.BlockSpec((1, H), lambda i: (0, 0)),
            pl.BlockSpec((1, H), lambda i: (0, 0)),
        ],
        out_specs=pl.BlockSpec((N + 1, FB, H), lambda i: (0, i, 0)),
        out_shape=jax.ShapeDtypeStruct((N + 1, B, H), jnp.float32),
    )(rel.reshape(B, N, H), ent.reshape(B, N, H), graph_token, g2, b2)
    return out_t.transpose(1, 0, 2)


# speedup vs baseline: 1.0663x; 1.0663x over previous
import functools

import jax
import jax.numpy as jnp
from jax import lax
from jax.experimental import pallas as pl
from jax.experimental.pallas import tpu as pltpu
from jax.experimental.pallas import tpu_sc as plsc

B, N, K, H = 16, 1024, 4, 128
R = 512
T = B * N
NC, NS = 2, 16
NW = NC * NS
TPW = T // NW

G_TOK = 128
NG = TPW // G_TOK


def _sc_body(idx_hbm, table_hbm, ent_hbm, idx_v,
             e0, e1, b0, b1, b2, b3, b4,
             se0, se1, sb0, sb1, sb2, sb3, sb4, ss0, ss1):
    wid = lax.axis_index("s") * NC + lax.axis_index("c")
    pltpu.sync_copy(idx_hbm.at[pl.ds(wid * (TPW * K // 128), TPW * K // 128)],
                    idx_v)

    def gath(g, k, buf, sem):
        pltpu.make_async_copy(table_hbm.at[idx_v.at[4 * g + k]], buf,
                              sem).start()

    def wait(buf, sem):
        pltpu.make_async_copy(table_hbm.at[idx_v.at[0]], buf, sem).wait()

    def reduce(e, a, b, c):
        def red(u, _):
            for dt in range(4):
                t = 4 * u + dt
                for j in range(H // 16):
                    s = pl.ds(j * 16, 16)
                    e[t, s] = e[t, s] + a[t, s] + b[t, s] + c[t, s]
            return 0
        lax.fori_loop(0, G_TOK // 4, red, 0)

    def store(g, e, sem):
        pltpu.make_async_copy(
            e, ent_hbm.at[pl.ds(wid * TPW + g * G_TOK, G_TOK)], sem).start()

    def swait(e, g, sem):
        pltpu.make_async_copy(
            e, ent_hbm.at[pl.ds(wid * TPW + g * G_TOK, G_TOK)], sem).wait()

    gath(0, 0, e0, se0)
    gath(0, 1, b0, sb0)
    gath(0, 2, b1, sb1)
    gath(0, 3, b2, sb2)
    wait(e0, se0); wait(b0, sb0); wait(b1, sb1); wait(b2, sb2)
    gath(1, 0, e1, se1)
    gath(1, 1, b3, sb3)
    gath(1, 2, b4, sb4)
    reduce(e0, b0, b1, b2)
    store(0, e0, ss0)
    gath(1, 3, b0, sb0)
    gath(2, 1, b1, sb1)
    gath(2, 2, b2, sb2)
    wait(e1, se1); wait(b3, sb3); wait(b4, sb4); wait(b0, sb0)
    reduce(e1, b3, b4, b0)
    store(1, e1, ss1)
    swait(e0, 0, ss0)
    gath(2, 0, e0, se0)
    gath(2, 3, b3, sb3)
    gath(3, 1, b4, sb4)
    wait(e0, se0); wait(b1, sb1); wait(b2, sb2); wait(b3, sb3)
    reduce(e0, b1, b2, b3)
    store(2, e0, ss0)
    gath(3, 2, b0, sb0)
    gath(3, 3, b1, sb1)
    swait(e1, 1, ss1)
    gath(3, 0, e1, se1)
    wait(e1, se1); wait(b4, sb4); wait(b0, sb0); wait(b1, sb1)
    reduce(e1, b4, b0, b1)
    store(3, e1, ss1)
    swait(e0, 2, ss0)
    swait(e1, 3, ss1)


_sc_call = functools.partial(
    pl.kernel,
    mesh=plsc.VectorSubcoreMesh(core_axis_name="c", subcore_axis_name="s"),
    out_type=jax.ShapeDtypeStruct((T, H), jnp.float32),
    scratch_types=[
        pltpu.VMEM((TPW * K // 128, 128), jnp.int32),
        pltpu.VMEM((G_TOK, H), jnp.float32),
        pltpu.VMEM((G_TOK, H), jnp.float32),
        pltpu.VMEM((G_TOK, H), jnp.float32),
        pltpu.VMEM((G_TOK, H), jnp.float32),
        pltpu.VMEM((G_TOK, H), jnp.float32),
        pltpu.VMEM((G_TOK, H), jnp.float32),
        pltpu.VMEM((G_TOK, H), jnp.float32),
        pltpu.SemaphoreType.DMA,
        pltpu.SemaphoreType.DMA,
        pltpu.SemaphoreType.DMA,
        pltpu.SemaphoreType.DMA,
        pltpu.SemaphoreType.DMA,
        pltpu.SemaphoreType.DMA,
        pltpu.SemaphoreType.DMA,
        pltpu.SemaphoreType.DMA,
        pltpu.SemaphoreType.DMA,
    ],
)(_sc_body)


TBLK = 1024


def _rel_body(rt_ref, fw_ref, link_ref, out_ref, w2_ref):
    @pl.when(pl.program_id(0) == 0)
    def _():
        w2_ref[...] = jnp.dot(rt_ref[...], fw_ref[...],
                              preferred_element_type=jnp.float32)

    ls = (link_ref[:, 0, :] + link_ref[:, 1, :]
          + link_ref[:, 2, :] + link_ref[:, 3, :])
    out_ref[...] = jnp.dot(ls, w2_ref[...],
                           preferred_element_type=jnp.float32
                           ).astype(jnp.bfloat16)


FB = 8


def _fin_body(rel_ref, ent_ref, gt_ref, g_ref, b_ref, out_ref):
    acc = rel_ref[...].astype(jnp.float32) + ent_ref[...]
    mu = jnp.mean(acc, axis=-1, keepdims=True)
    d = acc - mu
    var = jnp.mean(d * d, axis=-1, keepdims=True)
    y = d * lax.rsqrt(var + 1e-6) * g_ref[...] + b_ref[...]
    out_ref[1:, :, :] = jnp.transpose(y, (1, 0, 2))
    out_ref[0:1, :, :] = jnp.broadcast_to(
        gt_ref[...].reshape(1, 1, H), (1, FB, H))


def kernel(x, in_degree, out_degree, link, length, entity_table,
           in_deg_table, out_deg_table, rel_table, ffn_W,
           ln_gamma, ln_beta, graph_token):
    idx = (x.astype(jnp.int32)
           .reshape(B, N // 128, 128, K)
           .transpose(0, 1, 3, 2)
           .reshape(T * K // 128, 128))
    link_flat = link.reshape(T, K, R)
    ent = _sc_call(idx, entity_table)

    rel = pl.pallas_call(
        _rel_body,
        grid=(T // TBLK,),
        in_specs=[
            pl.BlockSpec((R, H), lambda i: (0, 0)),
            pl.BlockSpec((H, H), lambda i: (0, 0)),
            pl.BlockSpec((TBLK, K, R), lambda i: (i, 0, 0)),
        ],
        out_specs=pl.BlockSpec((TBLK, H), lambda i: (i, 0)),
        out_shape=jax.ShapeDtypeStruct((T, H), jnp.bfloat16),
        scratch_shapes=[pltpu.VMEM((R, H), jnp.float32)],
    )(rel_table, ffn_W, link_flat)

    g2 = ln_gamma.reshape(1, H)
    b2 = ln_beta.reshape(1, H)
    out_t = pl.pallas_call(
        _fin_body,
        grid=(B // FB,),
        in_specs=[
            pl.BlockSpec((FB, N, H), lambda i: (i, 0, 0)),
            pl.BlockSpec((FB, N, H), lambda i: (i, 0, 0)),
            pl.BlockSpec((1, H), lambda i: (0, 0)),
            pl.BlockSpec((1, H), lambda i: (0, 0)),
            pl.BlockSpec((1, H), lambda i: (0, 0)),
        ],
        out_specs=pl.BlockSpec((N + 1, FB, H), lambda i: (0, i, 0)),
        out_shape=jax.ShapeDtypeStruct((N + 1, B, H), jnp.float32),
    )(rel.reshape(B, N, H), ent.reshape(B, N, H), graph_token, g2, b2)
    return out_t.transpose(1, 0, 2)
